# Initial kernel scaffold; baseline (speedup 1.0000x reference)
import jax
import jax.numpy as jnp
from jax.experimental import pallas as pl


def kernel(x, edge_attr, F, pos, r, d, mean_pos, params, edge_index, batch):
    raise NotImplementedError("write your pallas kernel here")


# speedup vs baseline: 5.5343x; 5.5343x over previous
import functools

import jax
import jax.numpy as jnp
from jax import lax
from jax.experimental import pallas as pl
from jax.experimental.pallas import tpu as pltpu
from jax.experimental.pallas import tpu_sc as plsc

N = 10000
E = 320000
B = 4
MSG = 128

_NW = 32
_PER_W = E // _NW
_C = 200

_BLK = 1280
_GRID = E // _BLK

_NPAD = 10240


def _proj_body(x_ref, wj_ref, wi_ref, xa_ref, xb_ref):
    x = x_ref[...]
    xa_ref[...] = jnp.dot(x, wj_ref[...], preferred_element_type=jnp.float32)
    xb_ref[...] = jnp.dot(x, wi_ref[...], preferred_element_type=jnp.float32)


def _project(x, wj, wi):
    blk = 2000
    return pl.pallas_call(
        _proj_body,
        grid=(N // blk,),
        in_specs=[
            pl.BlockSpec((blk, 128), lambda i: (i, 0)),
            pl.BlockSpec((128, 128), lambda i: (0, 0)),
            pl.BlockSpec((128, 128), lambda i: (0, 0)),
        ],
        out_specs=[
            pl.BlockSpec((blk, 128), lambda i: (i, 0)),
            pl.BlockSpec((blk, 128), lambda i: (i, 0)),
        ],
        out_shape=[
            jax.ShapeDtypeStruct((N, 128), jnp.float32),
            jax.ShapeDtypeStruct((N, 128), jnp.float32),
        ],
    )(x, wj, wi)


def _gather_body(xa_hbm, xb_hbm, src_hbm, dst_hbm, g_hbm,
                 idx_a, idx_b, rows_a, rows_b, sem_a, sem_b):
    wid = lax.axis_index("s") * 2 + lax.axis_index("c")
    base = wid * _PER_W

    def chunk(k, carry):
        off = base + k * _C
        pltpu.sync_copy(src_hbm.at[pl.ds(off, _C)], idx_a)
        pltpu.sync_copy(dst_hbm.at[pl.ds(off, _C)], idx_b)
        ca = pltpu.async_copy(xa_hbm.at[idx_a], rows_a, sem_a)
        cb = pltpu.async_copy(xb_hbm.at[idx_b], rows_b, sem_b)
        ca.wait()
        cb.wait()

        def addrow(i, c2):
            for j in range(MSG // 16):
                s = pl.ds(j * 16, 16)
                rows_a[i, s] = rows_a[i, s] + rows_b[i, s]
            return c2

        lax.fori_loop(0, _C, addrow, 0)
        pltpu.sync_copy(rows_a, g_hbm.at[pl.ds(off, _C)])
        return carry

    lax.fori_loop(0, _PER_W // _C, chunk, 0)


def _gather_add(xa, xb, src, dst):
    mesh = plsc.VectorSubcoreMesh(core_axis_name="c", subcore_axis_name="s")
    f = pl.kernel(
        _gather_body,
        mesh=mesh,
        out_type=jax.ShapeDtypeStruct((E, MSG), jnp.float32),
        scratch_types=[
            pltpu.VMEM((_C,), jnp.int32),
            pltpu.VMEM((_C,), jnp.int32),
            pltpu.VMEM((_C, MSG), jnp.float32),
            pltpu.VMEM((_C, MSG), jnp.float32),
            pltpu.SemaphoreType.DMA,
            pltpu.SemaphoreType.DMA,
        ],
    )
    return f(xa, xb, src, dst)


def _softplus(x):
    return jnp.maximum(x, 0.0) + jnp.log1p(jnp.exp(-jnp.abs(x)))


def _main_body(g_ref, feat_ref, src_ref, batch_ref, f44_ref, wp_ref, wea_ref,
               bias_ref, en1_ref, en2_ref, en3_ref, p1_ref, p2_ref, p3_ref,
               d1_ref, d2_ref, d3_ref, out_ref):
    step = pl.program_id(0)

    @pl.when(step == 0)
    def _():
        out_ref[...] = jnp.zeros_like(out_ref)

    batch = batch_ref[...]
    src = src_ref[...]
    off1 = jnp.sum((batch < 1).astype(jnp.int32))
    off2 = jnp.sum((batch < 2).astype(jnp.int32))
    off3 = jnp.sum((batch < 3).astype(jnp.int32))
    eg = ((src >= off1).astype(jnp.int32) + (src >= off2).astype(jnp.int32)
          + (src >= off3).astype(jnp.int32))

    onehot8 = (eg == lax.broadcasted_iota(jnp.int32, (1, 8), 1)).astype(
        jnp.float32)
    fe = jnp.dot(onehot8, f44_ref[...], preferred_element_type=jnp.float32)

    feat = feat_ref[...]
    ri0 = feat[:, 0:1]
    ri1 = feat[:, 1:2]
    di = feat[:, 2:3]
    r0 = fe[:, 0:1] * ri0 + fe[:, 1:2] * ri1
    r1 = fe[:, 2:3] * ri0 + fe[:, 3:4] * ri1
    dd = jnp.sqrt(r0 * r0 + r1 * r1)

    wp = wp_ref[...]
    b = bias_ref[...]
    pre = (g_ref[...]
           + r0 * wp[0:1, :] + r1 * wp[1:2, :] + dd * wp[2:3, :]
           + ri0 * wp[3:4, :] + ri1 * wp[4:5, :] + di * wp[5:6, :]
           + jnp.dot(feat[:, 3:19], wea_ref[...],
                     preferred_element_type=jnp.float32)
           + b[0:1, :])
    m = _softplus(pre)

    h = _softplus(jnp.dot(m, en1_ref[...], preferred_element_type=jnp.float32)
                  + b[1:2, 0:64])
    h = _softplus(jnp.dot(h, en2_ref[...], preferred_element_type=jnp.float32)
                  + b[2:3, 0:64])
    en = jnp.dot(h, en3_ref[...], preferred_element_type=jnp.float32) + b[3:4, 0:1]

    hp = _softplus(jnp.dot(m, p1_ref[...], preferred_element_type=jnp.float32)
                   + b[4:5, :])
    hp = _softplus(jnp.dot(hp, p2_ref[...], preferred_element_type=jnp.float32)
                   + b[5:6, :])
    pp = jnp.dot(hp, p3_ref[...], preferred_element_type=jnp.float32) + b[6:7, 0:4]

    hd = _softplus(jnp.dot(m, d1_ref[...], preferred_element_type=jnp.float32)
                   + b[7:8, :])
    hd = _softplus(jnp.dot(hd, d2_ref[...], preferred_element_type=jnp.float32)
                   + b[8:9, :])
    pd = jnp.dot(hd, d3_ref[...], preferred_element_type=jnp.float32) + b[9:10, 0:16]

    ones = jnp.ones_like(en)
    zeros = jnp.zeros((en.shape[0], 128 - 22), jnp.float32)
    vals = jnp.concatenate([en, pp, pd, ones, zeros], axis=1)
    part = lax.dot_general(onehot8, vals, (((0,), (0,)), ((), ())),
                           preferred_element_type=jnp.float32)
    out_ref[...] += part

    @pl.when(step == _GRID - 1)
    def _():
        acc = out_ref[...]
        cnt = jnp.maximum(acc[:, 21:22], 1.0)
        col = lax.broadcasted_iota(jnp.int32, (8, 128), 1)
        div = jnp.logical_and(col >= 1, col <= 20)
        out_ref[...] = jnp.where(div, acc / cnt, acc)


def _main(g, feat, src2d, batch2d, f44, wp, wea, bias, en1, en2, en3,
          p1, p2, p3, d1, d2, d3):
    full = lambda shape: pl.BlockSpec(shape, lambda i: (0,) * len(shape))
    return pl.pallas_call(
        _main_body,
        grid=(_GRID,),
        in_specs=[
            pl.BlockSpec((_BLK, 128), lambda i: (i, 0)),
            pl.BlockSpec((_BLK, 19), lambda i: (i, 0)),
            pl.BlockSpec((_BLK, 1), lambda i: (i, 0)),
            full((80, 128)),
            full((8, 4)),
            full((8, 128)),
            full((16, 128)),
            full((16, 128)),
            full((128, 64)),
            full((64, 64)),
            full((64, 1)),
            full((128, 128)),
            full((128, 128)),
            full((128, 4)),
            full((128, 128)),
            full((128, 128)),
            full((128, 16)),
        ],
        out_specs=pl.BlockSpec((8, 128), lambda i: (0, 0)),
        out_shape=jax.ShapeDtypeStruct((8, 128), jnp.float32),
    )(g, feat, src2d, batch2d, f44, wp, wea, bias, en1, en2, en3,
      p1, p2, p3, d1, d2, d3)


def kernel(x, edge_attr, F, pos, r, d, mean_pos, params, edge_index, batch):
    p = params
    msg_W = p['msg_W']
    wj = msg_W[0:128]
    wi = msg_W[128:256]
    wp = jnp.zeros((8, 128), jnp.float32).at[0:6].set(msg_W[256:262])
    wea = jnp.zeros((16, 128), jnp.float32).at[0:16].set(msg_W[262:278])

    src = edge_index[0].astype(jnp.int32)
    dst = edge_index[1].astype(jnp.int32)
    feat = jnp.concatenate([r, d, edge_attr], axis=1)
    src2d = src.reshape(E, 1)

    batch2d = jnp.concatenate(
        [batch.astype(jnp.int32), jnp.full((_NPAD - N,), B, jnp.int32)]
    ).reshape(80, 128)

    f44 = jnp.zeros((8, 4), jnp.float32).at[0:4].set(F.reshape(4, 4))

    bias = jnp.zeros((16, 128), jnp.float32)
    bias = bias.at[0, :].set(p['msg_b'])
    bias = bias.at[1, 0:64].set(p['en1_b'])
    bias = bias.at[2, 0:64].set(p['en2_b'])
    bias = bias.at[3, 0:1].set(p['en3_b'])
    bias = bias.at[4, :].set(p['P1_b'])
    bias = bias.at[5, :].set(p['P2_b'])
    bias = bias.at[6, 0:4].set(p['P3_b'])
    bias = bias.at[7, :].set(p['D1_b'])
    bias = bias.at[8, :].set(p['D2_b'])
    bias = bias.at[9, 0:16].set(p['D3_b'])

    xa, xb = _project(x, wj, wi)
    g = _gather_add(xa, xb, src, dst)
    res = _main(g, feat, src2d, batch2d, f44, wp, wea, bias,
                p['en1_W'], p['en2_W'], p['en3_W'],
                p['P1_W'], p['P2_W'], p['P3_W'],
                p['D1_W'], p['D2_W'], p['D3_W'])
    return res[0:4, 0:21]
